# Initial kernel scaffold; baseline (speedup 1.0000x reference)
import jax
import jax.numpy as jnp
from jax.experimental import pallas as pl


def kernel(scores, score_feats, proposal_ids, point_ids):
    raise NotImplementedError("write your pallas kernel here")


# speedup vs baseline: 3.9727x; 3.9727x over previous
import functools

import jax
import jax.numpy as jnp
from jax import lax
from jax.experimental import pallas as pl
from jax.experimental.pallas import tpu as pltpu
from jax.experimental.pallas import tpu_sc as plsc

P = 512
N_POINTS = 32768
M = 262144
FEAT_DIM = 16
NMS_THRESH = 0.25
NUM_PROPOSAL = 256

_NSUB = 16
_PAIRS_PER_TILE = M // _NSUB
_WORDS = P * N_POINTS
_WORDS_PER_TILE = _WORDS // _NSUB
_ZCHUNK = 16384
_NZ = _WORDS_PER_TILE // _ZCHUNK


def _sc_scatter_body(pids_hbm, pts_hbm, mask_hbm,
                     pids_v, pts_v, idx_v, ones_v, zero_v, zsem, ssem):
    s = lax.axis_index("s")

    def fill(i, c):
        sl = pl.ds(i * 16, 16)
        zero_v[sl] = jnp.zeros((16,), jnp.float32)
        ones_v[sl] = jnp.ones((16,), jnp.float32)
        return c

    lax.fori_loop(0, _PAIRS_PER_TILE // 16, fill, 0)

    base = s * _WORDS_PER_TILE
    cps = [pltpu.async_copy(zero_v, mask_hbm.at[pl.ds(base + d * _ZCHUNK, _ZCHUNK)], zsem)
           for d in range(_NZ)]
    for cp in cps:
        cp.wait()
    plsc.subcore_barrier()

    mbase = s * _PAIRS_PER_TILE
    pltpu.sync_copy(pids_hbm.at[pl.ds(mbase, _PAIRS_PER_TILE)], pids_v)
    pltpu.sync_copy(pts_hbm.at[pl.ds(mbase, _PAIRS_PER_TILE)], pts_v)

    def cidx(i, c):
        sl = pl.ds(i * 16, 16)
        idx_v[sl] = pids_v[sl] * N_POINTS + pts_v[sl]
        return c

    lax.fori_loop(0, _PAIRS_PER_TILE // 16, cidx, 0)
    pltpu.async_copy(ones_v, mask_hbm.at[idx_v], ssem).wait()


def _build_mask(proposal_ids, point_ids):
    mesh = plsc.VectorSubcoreMesh(core_axis_name="c", subcore_axis_name="s",
                                  num_cores=1)
    fn = pl.kernel(
        _sc_scatter_body,
        out_type=jax.ShapeDtypeStruct((_WORDS,), jnp.float32),
        mesh=mesh,
        scratch_types=[
            pltpu.VMEM((_PAIRS_PER_TILE,), jnp.int32),
            pltpu.VMEM((_PAIRS_PER_TILE,), jnp.int32),
            pltpu.VMEM((_PAIRS_PER_TILE,), jnp.int32),
            pltpu.VMEM((_PAIRS_PER_TILE,), jnp.float32),
            pltpu.VMEM((_ZCHUNK,), jnp.float32),
            pltpu.SemaphoreType.DMA,
            pltpu.SemaphoreType.DMA,
        ],
    )
    return fn(proposal_ids, point_ids)


_KBLK = 4096
_KSTEPS = N_POINTS // _KBLK


def _accurate_sigmoid(x):
    z = jnp.clip(-x * 1.4426950408889634, -126.0, 126.0)
    n = jnp.floor(z + 0.5)
    f = z - n
    p = (((((1.535336188319500e-4 * f + 1.339887440266574e-3) * f
            + 9.618437357674640e-3) * f + 5.550332471162809e-2) * f
          + 2.402264791363012e-1) * f + 6.931472028550421e-1) * f + 1.0
    sc = lax.bitcast_convert_type((n.astype(jnp.int32) + 127) << 23,
                                  jnp.float32)
    return 1.0 / (1.0 + p * sc)


def _tc_body(mask_ref, s_ref, f_ref, keep_ref, vals_ref, feats_ref,
             acc_ref, ious_ref):
    k = pl.program_id(0)

    @pl.when(k == 0)
    def _():
        acc_ref[...] = jnp.zeros((P, P), jnp.float32)

    a = mask_ref[...].astype(jnp.bfloat16)
    acc_ref[...] += lax.dot_general(
        a, a, (((1,), (1,)), ((), ())), preferred_element_type=jnp.float32)

    @pl.when(k == _KSTEPS - 1)
    def _():
        inter = acc_ref[...]
        iota_r = lax.broadcasted_iota(jnp.int32, (P, P), 1)
        iota_c = lax.broadcasted_iota(jnp.int32, (P, P), 0)
        eye = (iota_r == iota_c).astype(jnp.float32)

        pn_c = jnp.sum(inter * eye, axis=1, keepdims=True)
        pn_r = jnp.sum(inter * eye, axis=0, keepdims=True)
        union = pn_c + pn_r - inter
        ious = inter / (union + 1e-8)

        s_row = _accurate_sigmoid(s_ref[...])
        s_col = jnp.sum(eye * s_row, axis=1, keepdims=True)

        cmp = ((s_col > s_row) |
               ((s_col == s_row) & (iota_c < iota_r))).astype(jnp.float32)
        rank_r = jnp.sum(cmp, axis=0, keepdims=True)
        rank_c = jnp.sum(eye * rank_r, axis=1, keepdims=True)

        iota1_r = lax.broadcasted_iota(jnp.int32, (1, P), 1).astype(jnp.float32)
        iota1_c = lax.broadcasted_iota(jnp.int32, (P, 1), 0).astype(jnp.float32)
        perm = (iota1_c == rank_r).astype(jnp.float32)
        perm_t = (rank_c == iota1_r).astype(jnp.float32)

        dn = (((1,), (0,)), ((), ()))
        hi = lax.Precision.HIGHEST
        ious_ref[...] = lax.dot_general(
            perm, lax.dot_general(ious, perm_t, dn, precision=hi,
                                  preferred_element_type=jnp.float32),
            dn, precision=hi, preferred_element_type=jnp.float32)

        def body(r, keep):
            row = ious_ref[pl.ds(r, 1), :]
            keep_r = jnp.sum(keep * (iota1_r == r.astype(jnp.float32)))
            sup = ((row > NMS_THRESH) &
                   (iota1_r > r.astype(jnp.float32))).astype(jnp.float32)
            factor = jnp.where(keep_r > 0.5, 1.0, 0.0)
            return keep * (1.0 - sup * factor)

        keep_s = lax.fori_loop(0, P, body, jnp.ones((1, P), jnp.float32))
        keep_o = lax.dot_general(keep_s, perm, dn, precision=hi,
                                 preferred_element_type=jnp.float32)
        keep_ref[...] = keep_o

        masked_r = jnp.where(keep_o > 0.5, s_row, -1.0)
        masked_c = jnp.sum(eye * masked_r, axis=1, keepdims=True)
        cmp2 = ((masked_c > masked_r) |
                ((masked_c == masked_r) & (iota_c < iota_r))).astype(jnp.float32)
        rank2_r = jnp.sum(cmp2, axis=0, keepdims=True)
        perm2 = (iota1_c == rank2_r).astype(jnp.float32)

        vals = jnp.sum(perm2 * masked_r, axis=1, keepdims=True)
        vals_ref[...] = vals[:NUM_PROPOSAL, :]
        feats = lax.dot_general(perm2, f_ref[...], dn, precision=hi,
                                preferred_element_type=jnp.float32)
        feats_ref[...] = feats[:NUM_PROPOSAL, :]


def _postprocess(mask, scores_row, score_feats):
    return pl.pallas_call(
        _tc_body,
        grid=(_KSTEPS,),
        in_specs=[
            pl.BlockSpec((P, _KBLK), lambda k: (0, k)),
            pl.BlockSpec((1, P), lambda k: (0, 0)),
            pl.BlockSpec((P, FEAT_DIM), lambda k: (0, 0)),
        ],
        out_specs=[
            pl.BlockSpec((1, P), lambda k: (0, 0)),
            pl.BlockSpec((NUM_PROPOSAL, 1), lambda k: (0, 0)),
            pl.BlockSpec((NUM_PROPOSAL, FEAT_DIM), lambda k: (0, 0)),
        ],
        out_shape=[
            jax.ShapeDtypeStruct((1, P), jnp.float32),
            jax.ShapeDtypeStruct((NUM_PROPOSAL, 1), jnp.float32),
            jax.ShapeDtypeStruct((NUM_PROPOSAL, FEAT_DIM), jnp.float32),
        ],
        scratch_shapes=[
            pltpu.VMEM((P, P), jnp.float32),
            pltpu.VMEM((P, P), jnp.float32),
        ],
    )(mask, scores_row, score_feats)


def kernel(scores, score_feats, proposal_ids, point_ids):
    mask = _build_mask(proposal_ids, point_ids).reshape(P, N_POINTS)
    keep_f, vals, feats = _postprocess(mask, scores.reshape(1, P), score_feats)
    keep = keep_f.reshape(P) > 0.5
    return keep, vals.reshape(NUM_PROPOSAL), feats


# speedup vs baseline: 3.9915x; 1.0047x over previous
import functools

import jax
import jax.numpy as jnp
from jax import lax
from jax.experimental import pallas as pl
from jax.experimental.pallas import tpu as pltpu
from jax.experimental.pallas import tpu_sc as plsc

P = 512
N_POINTS = 32768
M = 262144
FEAT_DIM = 16
NMS_THRESH = 0.25
NUM_PROPOSAL = 256

_NSUB = 16
_PAIRS_PER_TILE = M // _NSUB
_WORDS = P * N_POINTS
_WORDS_PER_TILE = _WORDS // _NSUB
_ZCHUNK = 32768
_NZ = _WORDS_PER_TILE // _ZCHUNK


def _sc_scatter_body(pids_hbm, pts_hbm, mask_hbm,
                     pids_v, pts_v, idx_v, ones_v, zero_v, zsem, lsem, ssem):
    s = lax.axis_index("s")

    def fillz(i, c):
        zero_v[pl.ds(i * 16, 16)] = jnp.zeros((16,), jnp.float32)
        return c

    lax.fori_loop(0, _ZCHUNK // 16, fillz, 0)

    base = s * _WORDS_PER_TILE
    cps = [pltpu.async_copy(zero_v, mask_hbm.at[pl.ds(base + d * _ZCHUNK, _ZCHUNK)], zsem)
           for d in range(_NZ)]

    mbase = s * _PAIRS_PER_TILE
    lp = pltpu.async_copy(pids_hbm.at[pl.ds(mbase, _PAIRS_PER_TILE)], pids_v, lsem)
    lt = pltpu.async_copy(pts_hbm.at[pl.ds(mbase, _PAIRS_PER_TILE)], pts_v, lsem)

    def fillo(i, c):
        ones_v[pl.ds(i * 16, 16)] = jnp.ones((16,), jnp.float32)
        return c

    lax.fori_loop(0, _PAIRS_PER_TILE // 16, fillo, 0)
    lp.wait()
    lt.wait()

    def cidx(i, c):
        sl = pl.ds(i * 16, 16)
        idx_v[sl] = pids_v[sl] * N_POINTS + pts_v[sl]
        return c

    lax.fori_loop(0, _PAIRS_PER_TILE // 16, cidx, 0)

    for cp in cps:
        cp.wait()
    plsc.subcore_barrier()

    pltpu.async_copy(ones_v, mask_hbm.at[idx_v], ssem).wait()


def _build_mask(proposal_ids, point_ids):
    mesh = plsc.VectorSubcoreMesh(core_axis_name="c", subcore_axis_name="s",
                                  num_cores=1)
    fn = pl.kernel(
        _sc_scatter_body,
        out_type=jax.ShapeDtypeStruct((_WORDS,), jnp.float32),
        mesh=mesh,
        scratch_types=[
            pltpu.VMEM((_PAIRS_PER_TILE,), jnp.int32),
            pltpu.VMEM((_PAIRS_PER_TILE,), jnp.int32),
            pltpu.VMEM((_PAIRS_PER_TILE,), jnp.int32),
            pltpu.VMEM((_PAIRS_PER_TILE,), jnp.float32),
            pltpu.VMEM((_ZCHUNK,), jnp.float32),
            pltpu.SemaphoreType.DMA,
            pltpu.SemaphoreType.DMA,
            pltpu.SemaphoreType.DMA,
        ],
    )
    return fn(proposal_ids, point_ids)


_KBLK = 4096
_KSTEPS = N_POINTS // _KBLK


def _accurate_sigmoid(x):
    z = jnp.clip(-x * 1.4426950408889634, -126.0, 126.0)
    n = jnp.floor(z + 0.5)
    f = z - n
    p = (((((1.535336188319500e-4 * f + 1.339887440266574e-3) * f
            + 9.618437357674640e-3) * f + 5.550332471162809e-2) * f
          + 2.402264791363012e-1) * f + 6.931472028550421e-1) * f + 1.0
    sc = lax.bitcast_convert_type((n.astype(jnp.int32) + 127) << 23,
                                  jnp.float32)
    return 1.0 / (1.0 + p * sc)


def _tc_body(mask_ref, s_ref, f_ref, keep_ref, vals_ref, feats_ref,
             acc_ref, ious_ref):
    k = pl.program_id(0)

    @pl.when(k == 0)
    def _():
        acc_ref[...] = jnp.zeros((P, P), jnp.float32)

    a = mask_ref[...].astype(jnp.bfloat16)
    acc_ref[...] += lax.dot_general(
        a, a, (((1,), (1,)), ((), ())), preferred_element_type=jnp.float32)

    @pl.when(k == _KSTEPS - 1)
    def _():
        inter = acc_ref[...]
        iota_r = lax.broadcasted_iota(jnp.int32, (P, P), 1)
        iota_c = lax.broadcasted_iota(jnp.int32, (P, P), 0)
        eye = (iota_r == iota_c).astype(jnp.float32)

        pn_c = jnp.sum(inter * eye, axis=1, keepdims=True)
        pn_r = jnp.sum(inter * eye, axis=0, keepdims=True)
        union = pn_c + pn_r - inter
        ious = inter / (union + 1e-8)

        s_row = _accurate_sigmoid(s_ref[...])
        s_col = jnp.sum(eye * s_row, axis=1, keepdims=True)

        cmp = ((s_col > s_row) |
               ((s_col == s_row) & (iota_c < iota_r))).astype(jnp.float32)
        rank_r = jnp.sum(cmp, axis=0, keepdims=True)
        rank_c = jnp.sum(eye * rank_r, axis=1, keepdims=True)

        iota1_r = lax.broadcasted_iota(jnp.int32, (1, P), 1).astype(jnp.float32)
        iota1_c = lax.broadcasted_iota(jnp.int32, (P, 1), 0).astype(jnp.float32)
        perm = (iota1_c == rank_r).astype(jnp.float32)
        perm_t = (rank_c == iota1_r).astype(jnp.float32)

        dn = (((1,), (0,)), ((), ()))
        hi = lax.Precision.HIGHEST
        ious_ref[...] = lax.dot_general(
            perm, lax.dot_general(ious, perm_t, dn, precision=hi,
                                  preferred_element_type=jnp.float32),
            dn, precision=hi, preferred_element_type=jnp.float32)

        def body(r, keep):
            row = ious_ref[pl.ds(r, 1), :]
            keep_r = jnp.sum(keep * (iota1_r == r.astype(jnp.float32)))
            sup = ((row > NMS_THRESH) &
                   (iota1_r > r.astype(jnp.float32))).astype(jnp.float32)
            factor = jnp.where(keep_r > 0.5, 1.0, 0.0)
            return keep * (1.0 - sup * factor)

        keep_s = lax.fori_loop(0, P, body, jnp.ones((1, P), jnp.float32))
        keep_o = lax.dot_general(keep_s, perm, dn, precision=hi,
                                 preferred_element_type=jnp.float32)
        keep_ref[...] = keep_o

        masked_r = jnp.where(keep_o > 0.5, s_row, -1.0)
        masked_c = jnp.sum(eye * masked_r, axis=1, keepdims=True)
        cmp2 = ((masked_c > masked_r) |
                ((masked_c == masked_r) & (iota_c < iota_r))).astype(jnp.float32)
        rank2_r = jnp.sum(cmp2, axis=0, keepdims=True)
        perm2 = (iota1_c == rank2_r).astype(jnp.float32)

        vals = jnp.sum(perm2 * masked_r, axis=1, keepdims=True)
        vals_ref[...] = vals[:NUM_PROPOSAL, :]
        feats = lax.dot_general(perm2, f_ref[...], dn, precision=hi,
                                preferred_element_type=jnp.float32)
        feats_ref[...] = feats[:NUM_PROPOSAL, :]


def _postprocess(mask, scores_row, score_feats):
    return pl.pallas_call(
        _tc_body,
        grid=(_KSTEPS,),
        in_specs=[
            pl.BlockSpec((P, _KBLK), lambda k: (0, k)),
            pl.BlockSpec((1, P), lambda k: (0, 0)),
            pl.BlockSpec((P, FEAT_DIM), lambda k: (0, 0)),
        ],
        out_specs=[
            pl.BlockSpec((1, P), lambda k: (0, 0)),
            pl.BlockSpec((NUM_PROPOSAL, 1), lambda k: (0, 0)),
            pl.BlockSpec((NUM_PROPOSAL, FEAT_DIM), lambda k: (0, 0)),
        ],
        out_shape=[
            jax.ShapeDtypeStruct((1, P), jnp.float32),
            jax.ShapeDtypeStruct((NUM_PROPOSAL, 1), jnp.float32),
            jax.ShapeDtypeStruct((NUM_PROPOSAL, FEAT_DIM), jnp.float32),
        ],
        scratch_shapes=[
            pltpu.VMEM((P, P), jnp.float32),
            pltpu.VMEM((P, P), jnp.float32),
        ],
    )(mask, scores_row, score_feats)


def kernel(scores, score_feats, proposal_ids, point_ids):
    mask = _build_mask(proposal_ids, point_ids).reshape(P, N_POINTS)
    keep_f, vals, feats = _postprocess(mask, scores.reshape(1, P), score_feats)
    keep = keep_f.reshape(P) > 0.5
    return keep, vals.reshape(NUM_PROPOSAL), feats


# speedup vs baseline: 4.0495x; 1.0145x over previous
import functools

import jax
import jax.numpy as jnp
from jax import lax
from jax.experimental import pallas as pl
from jax.experimental.pallas import tpu as pltpu
from jax.experimental.pallas import tpu_sc as plsc

P = 512
N_POINTS = 32768
M = 262144
FEAT_DIM = 16
NMS_THRESH = 0.25
NUM_PROPOSAL = 256

_NSUB = 16
_PAIRS_PER_TILE = M // _NSUB
_WORDS = P * N_POINTS
_WORDS_PER_TILE = _WORDS // _NSUB
_ZCHUNK = 32768
_NZ = _WORDS_PER_TILE // _ZCHUNK


def _sc_scatter_body(pids_hbm, pts_hbm, mask_hbm,
                     pids_v, pts_v, idx_v, ones_v, zero_v, zsem, lsem, ssem):
    s = lax.axis_index("s")

    def fillz(i, c):
        zero_v[pl.ds(i * 16, 16)] = jnp.zeros((16,), jnp.float32)
        return c

    lax.fori_loop(0, _ZCHUNK // 16, fillz, 0, unroll=16)

    base = s * _WORDS_PER_TILE
    cps = [pltpu.async_copy(zero_v, mask_hbm.at[pl.ds(base + d * _ZCHUNK, _ZCHUNK)], zsem)
           for d in range(_NZ)]

    mbase = s * _PAIRS_PER_TILE
    lp = pltpu.async_copy(pids_hbm.at[pl.ds(mbase, _PAIRS_PER_TILE)], pids_v, lsem)
    lt = pltpu.async_copy(pts_hbm.at[pl.ds(mbase, _PAIRS_PER_TILE)], pts_v, lsem)

    def fillo(i, c):
        ones_v[pl.ds(i * 16, 16)] = jnp.ones((16,), jnp.float32)
        return c

    lax.fori_loop(0, _PAIRS_PER_TILE // 16, fillo, 0, unroll=16)
    lp.wait()
    lt.wait()

    def cidx(i, c):
        sl = pl.ds(i * 16, 16)
        idx_v[sl] = pids_v[sl] * N_POINTS + pts_v[sl]
        return c

    lax.fori_loop(0, _PAIRS_PER_TILE // 16, cidx, 0, unroll=16)

    for cp in cps:
        cp.wait()
    plsc.subcore_barrier()

    pltpu.async_copy(ones_v, mask_hbm.at[idx_v], ssem).wait()


def _build_mask(proposal_ids, point_ids):
    mesh = plsc.VectorSubcoreMesh(core_axis_name="c", subcore_axis_name="s",
                                  num_cores=1)
    fn = pl.kernel(
        _sc_scatter_body,
        out_type=jax.ShapeDtypeStruct((_WORDS,), jnp.float32),
        mesh=mesh,
        scratch_types=[
            pltpu.VMEM((_PAIRS_PER_TILE,), jnp.int32),
            pltpu.VMEM((_PAIRS_PER_TILE,), jnp.int32),
            pltpu.VMEM((_PAIRS_PER_TILE,), jnp.int32),
            pltpu.VMEM((_PAIRS_PER_TILE,), jnp.float32),
            pltpu.VMEM((_ZCHUNK,), jnp.float32),
            pltpu.SemaphoreType.DMA,
            pltpu.SemaphoreType.DMA,
            pltpu.SemaphoreType.DMA,
        ],
    )
    return fn(proposal_ids, point_ids)


_KBLK = 4096
_KSTEPS = N_POINTS // _KBLK


def _accurate_sigmoid(x):
    z = jnp.clip(-x * 1.4426950408889634, -126.0, 126.0)
    n = jnp.floor(z + 0.5)
    f = z - n
    p = (((((1.535336188319500e-4 * f + 1.339887440266574e-3) * f
            + 9.618437357674640e-3) * f + 5.550332471162809e-2) * f
          + 2.402264791363012e-1) * f + 6.931472028550421e-1) * f + 1.0
    sc = lax.bitcast_convert_type((n.astype(jnp.int32) + 127) << 23,
                                  jnp.float32)
    return 1.0 / (1.0 + p * sc)


def _tc_body(mask_ref, s_ref, f_ref, keep_ref, vals_ref, feats_ref,
             acc_ref, ious_ref):
    k = pl.program_id(0)

    @pl.when(k == 0)
    def _():
        acc_ref[...] = jnp.zeros((P, P), jnp.float32)

    a = mask_ref[...].astype(jnp.bfloat16)
    acc_ref[...] += lax.dot_general(
        a, a, (((1,), (1,)), ((), ())), preferred_element_type=jnp.float32)

    @pl.when(k == _KSTEPS - 1)
    def _():
        inter = acc_ref[...]
        iota_r = lax.broadcasted_iota(jnp.int32, (P, P), 1)
        iota_c = lax.broadcasted_iota(jnp.int32, (P, P), 0)
        eye = (iota_r == iota_c).astype(jnp.float32)

        pn_c = jnp.sum(inter * eye, axis=1, keepdims=True)
        pn_r = jnp.sum(inter * eye, axis=0, keepdims=True)
        union = pn_c + pn_r - inter
        ious = inter / (union + 1e-8)

        s_row = _accurate_sigmoid(s_ref[...])
        s_col = jnp.sum(eye * s_row, axis=1, keepdims=True)

        cmp = ((s_col > s_row) |
               ((s_col == s_row) & (iota_c < iota_r))).astype(jnp.float32)
        rank_r = jnp.sum(cmp, axis=0, keepdims=True)
        rank_c = jnp.sum(eye * rank_r, axis=1, keepdims=True)

        iota1_r = lax.broadcasted_iota(jnp.int32, (1, P), 1).astype(jnp.float32)
        iota1_c = lax.broadcasted_iota(jnp.int32, (P, 1), 0).astype(jnp.float32)
        perm = (iota1_c == rank_r).astype(jnp.float32)
        perm_t = (rank_c == iota1_r).astype(jnp.float32)

        dn = (((1,), (0,)), ((), ()))
        hi = lax.Precision.HIGHEST
        ious_ref[...] = lax.dot_general(
            perm, lax.dot_general(ious, perm_t, dn, precision=hi,
                                  preferred_element_type=jnp.float32),
            dn, precision=hi, preferred_element_type=jnp.float32)

        def body(r, keep):
            row = ious_ref[pl.ds(r, 1), :]
            keep_r = jnp.sum(keep * (iota1_r == r.astype(jnp.float32)))
            sup = ((row > NMS_THRESH) &
                   (iota1_r > r.astype(jnp.float32))).astype(jnp.float32)
            factor = jnp.where(keep_r > 0.5, 1.0, 0.0)
            return keep * (1.0 - sup * factor)

        keep_s = lax.fori_loop(0, P, body, jnp.ones((1, P), jnp.float32))
        keep_o = lax.dot_general(keep_s, perm, dn, precision=hi,
                                 preferred_element_type=jnp.float32)
        keep_ref[...] = keep_o

        masked_r = jnp.where(keep_o > 0.5, s_row, -1.0)
        masked_c = jnp.sum(eye * masked_r, axis=1, keepdims=True)
        cmp2 = ((masked_c > masked_r) |
                ((masked_c == masked_r) & (iota_c < iota_r))).astype(jnp.float32)
        rank2_r = jnp.sum(cmp2, axis=0, keepdims=True)
        perm2 = (iota1_c == rank2_r).astype(jnp.float32)

        vals = jnp.sum(perm2 * masked_r, axis=1, keepdims=True)
        vals_ref[...] = vals[:NUM_PROPOSAL, :]
        feats = lax.dot_general(perm2, f_ref[...], dn, precision=hi,
                                preferred_element_type=jnp.float32)
        feats_ref[...] = feats[:NUM_PROPOSAL, :]


def _postprocess(mask, scores_row, score_feats):
    return pl.pallas_call(
        _tc_body,
        grid=(_KSTEPS,),
        in_specs=[
            pl.BlockSpec((P, _KBLK), lambda k: (0, k)),
            pl.BlockSpec((1, P), lambda k: (0, 0)),
            pl.BlockSpec((P, FEAT_DIM), lambda k: (0, 0)),
        ],
        out_specs=[
            pl.BlockSpec((1, P), lambda k: (0, 0)),
            pl.BlockSpec((NUM_PROPOSAL, 1), lambda k: (0, 0)),
            pl.BlockSpec((NUM_PROPOSAL, FEAT_DIM), lambda k: (0, 0)),
        ],
        out_shape=[
            jax.ShapeDtypeStruct((1, P), jnp.float32),
            jax.ShapeDtypeStruct((NUM_PROPOSAL, 1), jnp.float32),
            jax.ShapeDtypeStruct((NUM_PROPOSAL, FEAT_DIM), jnp.float32),
        ],
        scratch_shapes=[
            pltpu.VMEM((P, P), jnp.float32),
            pltpu.VMEM((P, P), jnp.float32),
        ],
    )(mask, scores_row, score_feats)


def kernel(scores, score_feats, proposal_ids, point_ids):
    mask = _build_mask(proposal_ids, point_ids).reshape(P, N_POINTS)
    keep_f, vals, feats = _postprocess(mask, scores.reshape(1, P), score_feats)
    keep = keep_f.reshape(P) > 0.5
    return keep, vals.reshape(NUM_PROPOSAL), feats
